# Initial kernel scaffold; baseline (speedup 1.0000x reference)
import jax
import jax.numpy as jnp
from jax.experimental import pallas as pl


def kernel(adv_patch, face_to_edges_idx, edge_len):
    raise NotImplementedError("write your pallas kernel here")


# speedup vs baseline: 13.5501x; 13.5501x over previous
import functools

import jax
import jax.numpy as jnp
from jax import lax
from jax.experimental import pallas as pl
from jax.experimental.pallas import tpu as pltpu
from jax.experimental.pallas import tpu_sc as plsc

F = 100000
E = 150000
D = 192
L = 16
NC, NS = 2, 16
NW = NC * NS
B = 128
CHUNKS = 37
E_PER_W = B * CHUNKS
E_PAD = E_PER_W * NW


def _tv_kernel(patch_hbm, idx0_hbm, idx1_hbm, len_hbm, out_hbm,
               i0_v, i1_v, len_v, f1_v, f2_v, acc_v, sem0, sem1):
    wid = lax.axis_index("s") * NC + lax.axis_index("c")
    base = pl.multiple_of(wid * E_PER_W, B)

    def chunk_body(c, tot):
        off = pl.multiple_of(base + c * B, B)
        pltpu.sync_copy(idx0_hbm.at[pl.ds(off, B)], i0_v)
        pltpu.sync_copy(idx1_hbm.at[pl.ds(off, B)], i1_v)
        pltpu.sync_copy(len_hbm.at[pl.ds(off, B)], len_v)
        cp0 = pltpu.async_copy(patch_hbm.at[i0_v], f1_v, sem0)
        cp1 = pltpu.async_copy(patch_hbm.at[i1_v], f2_v, sem1)
        cp0.wait()
        cp1.wait()

        def group_body(g, t):
            w_blk = len_v[pl.ds(g * L, L)]
            for k in range(L):
                e = g * L + k
                acc = jnp.abs(f1_v[e, pl.ds(0, L)] - f2_v[e, pl.ds(0, L)])
                for j in range(1, D // L):
                    acc += jnp.abs(f1_v[e, pl.ds(j * L, L)]
                                   - f2_v[e, pl.ds(j * L, L)])
                t = t + w_blk[k] * acc
            return t

        return lax.fori_loop(0, B // L, group_body, tot)

    tot = lax.fori_loop(0, CHUNKS, chunk_body, jnp.zeros((L,), jnp.float32))
    acc_v[...] = tot
    pltpu.sync_copy(acc_v, out_hbm.at[wid])


@jax.jit
def kernel(adv_patch, face_to_edges_idx, edge_len):
    patch2d = adv_patch.reshape(F, D)
    idx = face_to_edges_idx.astype(jnp.int32)
    pad = E_PAD - E
    idx0 = jnp.pad(idx[:, 0], (0, pad))
    idx1 = jnp.pad(idx[:, 1], (0, pad))
    len_p = jnp.pad(edge_len, (0, pad))

    mesh = plsc.VectorSubcoreMesh(core_axis_name="c", subcore_axis_name="s")
    run = pl.kernel(
        _tv_kernel,
        mesh=mesh,
        compiler_params=pltpu.CompilerParams(use_tc_tiling_on_sc=False),
        out_type=jax.ShapeDtypeStruct((NW, L), jnp.float32),
        scratch_types=[
            pltpu.VMEM((B,), jnp.int32),
            pltpu.VMEM((B,), jnp.int32),
            pltpu.VMEM((B,), jnp.float32),
            pltpu.VMEM((B, D), jnp.float32),
            pltpu.VMEM((B, D), jnp.float32),
            pltpu.VMEM((L,), jnp.float32),
            pltpu.SemaphoreType.DMA,
            pltpu.SemaphoreType.DMA,
        ],
    )
    partials = run(patch2d, idx0, idx1, len_p)
    return jnp.sum(partials) / F
